# Initial kernel scaffold; baseline (speedup 1.0000x reference)
import jax
import jax.numpy as jnp
from jax.experimental import pallas as pl


def kernel(x, edge_index, Wq1, Wk1, Wv1, Ws1, Wq2, Wk2, Wv2, Ws2, bq1, bk1, bv1, bs1, bq2, bk2, bv2, bs2):
    raise NotImplementedError("write your pallas kernel here")


# speedup vs baseline: 6.6308x; 6.6308x over previous
import functools
import math

import jax
import jax.numpy as jnp
from jax import lax
from jax.experimental import pallas as pl
from jax.experimental.pallas import tpu as pltpu
from jax.experimental.pallas import tpu_sc as plsc

N = 10000
D = 128
E = 320000

NC = 2
NS = 16
NW = NC * NS
EPW = E // NW
C = 80
NCHUNK = EPW // C
G = 16
NGROUP = C // G
DROWS = (N + D - 1) // D
NACC = N + DROWS + 1
RPT = 640
RCHUNK = 80
NRC = RPT // RCHUNK


def _edge_kernel_body(q_hbm, kv_hbm, src_hbm, dst_hbm, out_hbm,
                      srcv, dstv, d2v, qr, kvr, msg2, acc_sh,
                      sem1, sem2):
    cid = lax.axis_index("c")
    sid = lax.axis_index("s")
    wid = sid * NC + cid
    zero16 = jnp.zeros((16,), jnp.float32)
    lanes = lax.iota(jnp.int32, 16)
    idx15 = jnp.full((16, 1), 15, jnp.int32)
    dnums = lax.GatherDimensionNumbers(
        offset_dims=(), collapsed_slice_dims=(0,), start_index_map=(0,))

    def _zrow(r, _):
        for j in range(D // 16):
            msg2[r, pl.ds(16 * j, 16)] = zero16
        return _
    lax.fori_loop(0, RCHUNK, _zrow, None)
    row0 = sid * RPT
    for t in range(NRC):
        r = row0 + t * RCHUNK

        @pl.when(r < NACC)
        def _():
            pltpu.sync_copy(msg2, acc_sh.at[pl.ds(r, RCHUNK)])

    plsc.subcore_barrier()

    def _chunk(t, _):
        base = wid * EPW + t * C
        pltpu.sync_copy(src_hbm.at[pl.ds(base, C)], srcv)
        pltpu.sync_copy(dst_hbm.at[pl.ds(base, C)], dstv)
        cq = pltpu.async_copy(q_hbm.at[dstv], qr, sem1)
        ckv = pltpu.async_copy(kv_hbm.at[srcv], kvr, sem2)
        cq.wait()
        ckv.wait()

        def _group(g, _):
            gb = g * G
            e16 = zero16
            for ii in range(G):
                i = gb + ii
                acc = qr[i, pl.ds(0, 16)] * kvr[i, pl.ds(0, 16)]
                for j in range(1, D // 16):
                    acc = acc + qr[i, pl.ds(16 * j, 16)] * kvr[i, pl.ds(16 * j, 16)]
                s = plsc.cumsum(acc)
                ab = lax.gather(s, idx15, dnums, (1,),
                                mode=lax.GatherScatterMode.PROMISE_IN_BOUNDS)
                eb = jnp.exp(ab)
                e16 = jnp.where(lanes == ii, eb, e16)
                for j in range(D // 16):
                    qr[i, pl.ds(16 * j, 16)] = eb * kvr[i, pl.ds(D + 16 * j, 16)]
            dst16 = dstv[pl.ds(gb, G)]
            d2v[pl.ds(gb, G)] = N + lax.shift_right_logical(dst16, 7)
            plsc.store_scatter(msg2, [gb + lanes, jnp.bitwise_and(dst16, 127)],
                               e16)
            return _
        lax.fori_loop(0, NGROUP, _group, None)

        pltpu.sync_copy(qr, acc_sh.at[dstv], add=True)
        pltpu.sync_copy(msg2, acc_sh.at[d2v], add=True)

        def _rezero(g, _):
            gb = g * G
            dst16 = dstv[pl.ds(gb, G)]
            plsc.store_scatter(msg2, [gb + lanes, jnp.bitwise_and(dst16, 127)],
                               zero16)
            return _
        lax.fori_loop(0, NGROUP, _rezero, None)
        return _
    lax.fori_loop(0, NCHUNK, _chunk, None)

    plsc.subcore_barrier()
    for t in range(NRC):
        r = row0 + t * RCHUNK

        @pl.when(r < NACC)
        def _():
            pltpu.sync_copy(acc_sh.at[pl.ds(r, RCHUNK)],
                            out_hbm.at[cid, pl.ds(r, RCHUNK)])


@functools.cache
def _edge_kernel():
    mesh = plsc.VectorSubcoreMesh(core_axis_name="c", subcore_axis_name="s")
    return pl.kernel(
        _edge_kernel_body,
        mesh=mesh,
        compiler_params=pltpu.CompilerParams(needs_layout_passes=False),
        out_type=jax.ShapeDtypeStruct((NC, NACC, D), jnp.float32),
        scratch_types=[
            pltpu.VMEM((C,), jnp.int32),
            pltpu.VMEM((C,), jnp.int32),
            pltpu.VMEM((C,), jnp.int32),
            pltpu.VMEM((C, D), jnp.float32),
            pltpu.VMEM((C, 2 * D), jnp.float32),
            pltpu.VMEM((C, D), jnp.float32),
            pltpu.VMEM_SHARED((NACC, D), jnp.float32),
            pltpu.SemaphoreType.DMA,
            pltpu.SemaphoreType.DMA,
        ],
    )


_BN = 1000


def _proj_body(x_ref, w_ref, b_ref, q_ref, kv_ref, s_ref):
    y = jnp.dot(x_ref[...], w_ref[...],
                preferred_element_type=jnp.float32) + b_ref[...]
    q_ref[...] = y[:, :D]
    kv_ref[...] = y[:, D:3 * D]
    s_ref[...] = y[:, 3 * D:]


_PROJ_OUT = [
    jax.ShapeDtypeStruct((N, D), jnp.float32),
    jax.ShapeDtypeStruct((N, 2 * D), jnp.float32),
    jax.ShapeDtypeStruct((N, D), jnp.float32),
]
_PROJ_OUT_SPECS = [
    pl.BlockSpec((_BN, D), lambda i: (i, 0)),
    pl.BlockSpec((_BN, 2 * D), lambda i: (i, 0)),
    pl.BlockSpec((_BN, D), lambda i: (i, 0)),
]


def _proj(x, w, b):
    return pl.pallas_call(
        _proj_body,
        grid=(N // _BN,),
        in_specs=[
            pl.BlockSpec((_BN, D), lambda i: (i, 0)),
            pl.BlockSpec((D, 4 * D), lambda i: (0, 0)),
            pl.BlockSpec((1, 4 * D), lambda i: (0, 0)),
        ],
        out_specs=_PROJ_OUT_SPECS,
        out_shape=_PROJ_OUT,
    )(x, w, b)


def _norm(acc_ref, den_ref):
    num = acc_ref[0] + acc_ref[1]
    den = den_ref[0] + den_ref[1]
    return num / (den + 1e-16)


def _comb_proj_body(acc_ref, den_ref, s_ref, w_ref, b_ref,
                    q_ref, kv_ref, sk_ref):
    h = jnp.maximum(_norm(acc_ref, den_ref) + s_ref[...], 0.0)
    y = jnp.dot(h, w_ref[...], preferred_element_type=jnp.float32) + b_ref[...]
    q_ref[...] = y[:, :D]
    kv_ref[...] = y[:, D:3 * D]
    sk_ref[...] = y[:, 3 * D:]


def _comb_proj(acc, den, s, w, b):
    return pl.pallas_call(
        _comb_proj_body,
        grid=(N // _BN,),
        in_specs=[
            pl.BlockSpec((NC, _BN, D), lambda i: (0, i, 0)),
            pl.BlockSpec((NC, _BN, 1), lambda i: (0, i, 0)),
            pl.BlockSpec((_BN, D), lambda i: (i, 0)),
            pl.BlockSpec((D, 4 * D), lambda i: (0, 0)),
            pl.BlockSpec((1, 4 * D), lambda i: (0, 0)),
        ],
        out_specs=_PROJ_OUT_SPECS,
        out_shape=_PROJ_OUT,
    )(acc, den, s, w, b)


def _final_body(acc_ref, den_ref, s_ref, o_ref):
    o_ref[...] = _norm(acc_ref, den_ref) + s_ref[...]


def _final(acc, den, s):
    return pl.pallas_call(
        _final_body,
        grid=(N // _BN,),
        in_specs=[
            pl.BlockSpec((NC, _BN, D), lambda i: (0, i, 0)),
            pl.BlockSpec((NC, _BN, 1), lambda i: (0, i, 0)),
            pl.BlockSpec((_BN, D), lambda i: (i, 0)),
        ],
        out_specs=pl.BlockSpec((_BN, D), lambda i: (i, 0)),
        out_shape=jax.ShapeDtypeStruct((N, D), jnp.float32),
    )(acc, den, s)


def _split_den(acc):
    den = acc[:, N:N + DROWS, :].reshape(NC, DROWS * D)[:, :N, None]
    return den


def kernel(x, edge_index, Wq1, Wk1, Wv1, Ws1, Wq2, Wk2, Wv2, Ws2,
           bq1, bk1, bv1, bs1, bq2, bk2, bv2, bs2):
    src = edge_index[0]
    dst = edge_index[1]
    scale = 1.0 / math.sqrt(D)
    w1 = jnp.concatenate([Wq1.T * scale, Wk1.T, Wv1.T, Ws1.T], axis=1)
    b1 = jnp.concatenate([bq1 * scale, bk1, bv1, bs1])[None, :]
    w2 = jnp.concatenate([Wq2.T * scale, Wk2.T, Wv2.T, Ws2.T], axis=1)
    b2 = jnp.concatenate([bq2 * scale, bk2, bv2, bs2])[None, :]

    q1, kv1, s1 = _proj(x, w1, b1)
    acc1 = _edge_kernel()(q1, kv1, src, dst)
    q2, kv2, s2 = _comb_proj(acc1, _split_den(acc1), s1, w2, b2)
    acc2 = _edge_kernel()(q2, kv2, src, dst)
    return _final(acc2, _split_den(acc2), s2)


# speedup vs baseline: 7.2953x; 1.1002x over previous
import functools
import math

import jax
import jax.numpy as jnp
from jax import lax
from jax.experimental import pallas as pl
from jax.experimental.pallas import tpu as pltpu
from jax.experimental.pallas import tpu_sc as plsc

N = 10000
D = 128
E = 320000

NC = 2
NS = 16
NW = NC * NS
C = 48
NCHUNK = 210
EPW = C * NCHUNK
EPAD = NW * EPW
G = 16
NGROUP = C // G
DROWS = (N + D - 1) // D
NACC = N + DROWS + 1
PAD_DST = NACC - 1
RPT = 640
RCHUNK = 40
NRC = RPT // RCHUNK


def _edge_kernel_body(q_hbm, kv_hbm, src_hbm, dst_hbm, out_hbm,
                      srcv0, srcv1, dstv0, dstv1, d2v,
                      qr0, qr1, kvr0, kvr1, msg2, acc_sh,
                      semq0, semq1, semk0, semk1):
    cid = lax.axis_index("c")
    sid = lax.axis_index("s")
    wid = sid * NC + cid
    zero16 = jnp.zeros((16,), jnp.float32)
    lanes = lax.iota(jnp.int32, 16)
    idx15 = jnp.full((16, 1), 15, jnp.int32)
    dnums = lax.GatherDimensionNumbers(
        offset_dims=(), collapsed_slice_dims=(0,), start_index_map=(0,))
    srcvs, dstvs = (srcv0, srcv1), (dstv0, dstv1)
    qrs, kvrs = (qr0, qr1), (kvr0, kvr1)
    semqs, semks = (semq0, semq1), (semk0, semk1)

    def _zrow(r, _):
        for j in range(D // 16):
            msg2[r, pl.ds(16 * j, 16)] = zero16
        return _
    lax.fori_loop(0, C, _zrow, None)
    row0 = sid * RPT
    for t in range(NRC):
        r = row0 + t * RCHUNK

        @pl.when(r < NACC)
        def _():
            pltpu.sync_copy(msg2.at[pl.ds(0, RCHUNK)],
                            acc_sh.at[pl.ds(r, RCHUNK)])

    plsc.subcore_barrier()

    def _prefetch(u, b):
        base = wid * EPW + u * C
        pltpu.sync_copy(src_hbm.at[pl.ds(base, C)], srcvs[b])
        pltpu.sync_copy(dst_hbm.at[pl.ds(base, C)], dstvs[b])
        pltpu.async_copy(q_hbm.at[dstvs[b]], qrs[b], semqs[b])
        pltpu.async_copy(kv_hbm.at[srcvs[b]], kvrs[b], semks[b])

    def _compute_scatter(b):
        qr, kvr, dstv = qrs[b], kvrs[b], dstvs[b]
        pltpu.make_async_copy(q_hbm.at[dstv], qr, semqs[b]).wait()
        pltpu.make_async_copy(kv_hbm.at[srcvs[b]], kvr, semks[b]).wait()

        def _group(g, _):
            gb = g * G
            e16 = zero16
            for ii in range(G):
                i = gb + ii
                acc = qr[i, pl.ds(0, 16)] * kvr[i, pl.ds(0, 16)]
                for j in range(1, D // 16):
                    acc = acc + qr[i, pl.ds(16 * j, 16)] * kvr[i, pl.ds(16 * j, 16)]
                s = plsc.cumsum(acc)
                ab = lax.gather(s, idx15, dnums, (1,),
                                mode=lax.GatherScatterMode.PROMISE_IN_BOUNDS)
                eb = jnp.exp(ab)
                e16 = jnp.where(lanes == ii, eb, e16)
                for j in range(D // 16):
                    qr[i, pl.ds(16 * j, 16)] = eb * kvr[i, pl.ds(D + 16 * j, 16)]
            dst16 = dstv[pl.ds(gb, G)]
            d2v[pl.ds(gb, G)] = N + lax.shift_right_logical(dst16, 7)
            plsc.store_scatter(msg2, [gb + lanes, jnp.bitwise_and(dst16, 127)],
                               e16)
            return _
        lax.fori_loop(0, NGROUP, _group, None)

        pltpu.sync_copy(qr, acc_sh.at[dstv], add=True)
        pltpu.sync_copy(msg2, acc_sh.at[d2v], add=True)

        def _rezero(g, _):
            gb = g * G
            dst16 = dstv[pl.ds(gb, G)]
            plsc.store_scatter(msg2, [gb + lanes, jnp.bitwise_and(dst16, 127)],
                               zero16)
            return _
        lax.fori_loop(0, NGROUP, _rezero, None)

    _prefetch(0, 0)

    def _outer(o, _):
        t = 2 * o
        _prefetch(t + 1, 1)
        _compute_scatter(0)

        @pl.when(t + 2 < NCHUNK)
        def _():
            _prefetch(t + 2, 0)
        _compute_scatter(1)
        return _
    lax.fori_loop(0, NCHUNK // 2, _outer, None)

    plsc.subcore_barrier()
    for t in range(NRC):
        r = row0 + t * RCHUNK

        @pl.when(r < NACC)
        def _():
            pltpu.sync_copy(acc_sh.at[pl.ds(r, RCHUNK)],
                            out_hbm.at[cid, pl.ds(r, RCHUNK)])


@functools.cache
def _edge_kernel():
    mesh = plsc.VectorSubcoreMesh(core_axis_name="c", subcore_axis_name="s")
    return pl.kernel(
        _edge_kernel_body,
        mesh=mesh,
        compiler_params=pltpu.CompilerParams(needs_layout_passes=False),
        out_type=jax.ShapeDtypeStruct((NC, NACC, D), jnp.float32),
        scratch_types=[
            pltpu.VMEM((C,), jnp.int32),
            pltpu.VMEM((C,), jnp.int32),
            pltpu.VMEM((C,), jnp.int32),
            pltpu.VMEM((C,), jnp.int32),
            pltpu.VMEM((C,), jnp.int32),
            pltpu.VMEM((C, D), jnp.float32),
            pltpu.VMEM((C, D), jnp.float32),
            pltpu.VMEM((C, 2 * D), jnp.float32),
            pltpu.VMEM((C, 2 * D), jnp.float32),
            pltpu.VMEM((C, D), jnp.float32),
            pltpu.VMEM_SHARED((NACC, D), jnp.float32),
            pltpu.SemaphoreType.DMA,
            pltpu.SemaphoreType.DMA,
            pltpu.SemaphoreType.DMA,
            pltpu.SemaphoreType.DMA,
        ],
    )


_BN = 1000


def _proj_body(x_ref, w_ref, b_ref, q_ref, kv_ref, s_ref):
    y = jnp.dot(x_ref[...], w_ref[...],
                preferred_element_type=jnp.float32) + b_ref[...]
    q_ref[...] = y[:, :D]
    kv_ref[...] = y[:, D:3 * D]
    s_ref[...] = y[:, 3 * D:]


_PROJ_OUT = [
    jax.ShapeDtypeStruct((N, D), jnp.float32),
    jax.ShapeDtypeStruct((N, 2 * D), jnp.float32),
    jax.ShapeDtypeStruct((N, D), jnp.float32),
]
_PROJ_OUT_SPECS = [
    pl.BlockSpec((_BN, D), lambda i: (i, 0)),
    pl.BlockSpec((_BN, 2 * D), lambda i: (i, 0)),
    pl.BlockSpec((_BN, D), lambda i: (i, 0)),
]


def _proj(x, w, b):
    return pl.pallas_call(
        _proj_body,
        grid=(N // _BN,),
        in_specs=[
            pl.BlockSpec((_BN, D), lambda i: (i, 0)),
            pl.BlockSpec((D, 4 * D), lambda i: (0, 0)),
            pl.BlockSpec((1, 4 * D), lambda i: (0, 0)),
        ],
        out_specs=_PROJ_OUT_SPECS,
        out_shape=_PROJ_OUT,
    )(x, w, b)


def _norm(acc_ref, den_ref):
    num = acc_ref[0] + acc_ref[1]
    den = den_ref[0] + den_ref[1]
    return num / (den + 1e-16)


def _comb_proj_body(acc_ref, den_ref, s_ref, w_ref, b_ref,
                    q_ref, kv_ref, sk_ref):
    h = jnp.maximum(_norm(acc_ref, den_ref) + s_ref[...], 0.0)
    y = jnp.dot(h, w_ref[...], preferred_element_type=jnp.float32) + b_ref[...]
    q_ref[...] = y[:, :D]
    kv_ref[...] = y[:, D:3 * D]
    sk_ref[...] = y[:, 3 * D:]


def _comb_proj(acc, den, s, w, b):
    return pl.pallas_call(
        _comb_proj_body,
        grid=(N // _BN,),
        in_specs=[
            pl.BlockSpec((NC, _BN, D), lambda i: (0, i, 0)),
            pl.BlockSpec((NC, _BN, 1), lambda i: (0, i, 0)),
            pl.BlockSpec((_BN, D), lambda i: (i, 0)),
            pl.BlockSpec((D, 4 * D), lambda i: (0, 0)),
            pl.BlockSpec((1, 4 * D), lambda i: (0, 0)),
        ],
        out_specs=_PROJ_OUT_SPECS,
        out_shape=_PROJ_OUT,
    )(acc, den, s, w, b)


def _final_body(acc_ref, den_ref, s_ref, o_ref):
    o_ref[...] = _norm(acc_ref, den_ref) + s_ref[...]


def _final(acc, den, s):
    return pl.pallas_call(
        _final_body,
        grid=(N // _BN,),
        in_specs=[
            pl.BlockSpec((NC, _BN, D), lambda i: (0, i, 0)),
            pl.BlockSpec((NC, _BN, 1), lambda i: (0, i, 0)),
            pl.BlockSpec((_BN, D), lambda i: (i, 0)),
        ],
        out_specs=pl.BlockSpec((_BN, D), lambda i: (i, 0)),
        out_shape=jax.ShapeDtypeStruct((N, D), jnp.float32),
    )(acc, den, s)


def _split_den(acc):
    den = acc[:, N:N + DROWS, :].reshape(NC, DROWS * D)[:, :N, None]
    return den


def kernel(x, edge_index, Wq1, Wk1, Wv1, Ws1, Wq2, Wk2, Wv2, Ws2,
           bq1, bk1, bv1, bs1, bq2, bk2, bv2, bs2):
    npad = EPAD - E
    src = jnp.concatenate([edge_index[0], jnp.zeros((npad,), jnp.int32)])
    dst = jnp.concatenate(
        [edge_index[1], jnp.full((npad,), PAD_DST, jnp.int32)])
    scale = 1.0 / math.sqrt(D)
    w1 = jnp.concatenate([Wq1.T * scale, Wk1.T, Wv1.T, Ws1.T], axis=1)
    b1 = jnp.concatenate([bq1 * scale, bk1, bv1, bs1])[None, :]
    w2 = jnp.concatenate([Wq2.T * scale, Wk2.T, Wv2.T, Ws2.T], axis=1)
    b2 = jnp.concatenate([bq2 * scale, bk2, bv2, bs2])[None, :]

    qpad = jnp.zeros((NACC - N, D), jnp.float32)
    q1, kv1, s1 = _proj(x, w1, b1)
    acc1 = _edge_kernel()(jnp.concatenate([q1, qpad]), kv1, src, dst)
    q2, kv2, s2 = _comb_proj(acc1, _split_den(acc1), s1, w2, b2)
    acc2 = _edge_kernel()(jnp.concatenate([q2, qpad]), kv2, src, dst)
    return _final(acc2, _split_den(acc2), s2)


# speedup vs baseline: 8.7251x; 1.1960x over previous
import functools
import math

import jax
import jax.numpy as jnp
from jax import lax
from jax.experimental import pallas as pl
from jax.experimental.pallas import tpu as pltpu
from jax.experimental.pallas import tpu_sc as plsc

N = 10000
D = 128
E = 320000

NC = 2
NS = 16
NW = NC * NS
C = 48
NCHUNK = 210
EPW = C * NCHUNK
EPAD = NW * EPW
G = 16
NGROUP = C // G
DROWS = (N + D - 1) // D
NACC = N + DROWS + 1
PAD_DST = NACC - 1
RPT = 640
RCHUNK = 40
NRC = RPT // RCHUNK


def _edge_kernel_body(q_hbm, kv_hbm, src_hbm, dst_hbm, out_hbm,
                      srcv0, srcv1, dstv0, dstv1, d2v,
                      qr0, qr1, kvr0, kvr1, msg2, acc_sh,
                      semq0, semq1, semk0, semk1):
    cid = lax.axis_index("c")
    sid = lax.axis_index("s")
    wid = sid * NC + cid
    zero16 = jnp.zeros((16,), jnp.float32)
    lanes = lax.iota(jnp.int32, 16)
    dnums = lax.GatherDimensionNumbers(
        offset_dims=(), collapsed_slice_dims=(0,), start_index_map=(0,))
    bfly = [jnp.reshape(jnp.bitwise_xor(lanes, 1 << k), (16, 1))
            for k in range(4)]
    splat = [jnp.full((16, 1), i, jnp.int32) for i in range(G)]
    srcvs, dstvs = (srcv0, srcv1), (dstv0, dstv1)
    qrs, kvrs = (qr0, qr1), (kvr0, kvr1)
    semqs, semks = (semq0, semq1), (semk0, semk1)

    def _zrow(r, _):
        for j in range(D // 16):
            msg2[r, pl.ds(16 * j, 16)] = zero16
        return _
    lax.fori_loop(0, C, _zrow, None)
    row0 = sid * RPT
    for t in range(NRC):
        r = row0 + t * RCHUNK

        @pl.when(r < NACC)
        def _():
            pltpu.sync_copy(msg2.at[pl.ds(0, RCHUNK)],
                            acc_sh.at[pl.ds(r, RCHUNK)])

    plsc.subcore_barrier()

    def _prefetch(u, b):
        base = wid * EPW + u * C
        pltpu.sync_copy(src_hbm.at[pl.ds(base, C)], srcvs[b])
        pltpu.sync_copy(dst_hbm.at[pl.ds(base, C)], dstvs[b])
        pltpu.async_copy(q_hbm.at[dstvs[b]], qrs[b], semqs[b])
        pltpu.async_copy(kv_hbm.at[srcvs[b]], kvrs[b], semks[b])

    def _compute_scatter(b):
        qr, kvr, dstv = qrs[b], kvrs[b], dstvs[b]
        pltpu.make_async_copy(q_hbm.at[dstv], qr, semqs[b]).wait()
        pltpu.make_async_copy(kv_hbm.at[srcvs[b]], kvr, semks[b]).wait()

        def _group(g, _):
            gb = g * G

            def _dot(ii, alpha16):
                i = gb + ii
                acc = qr[i, pl.ds(0, 16)] * kvr[i, pl.ds(0, 16)]
                for j in range(1, D // 16):
                    acc = acc + qr[i, pl.ds(16 * j, 16)] * kvr[i, pl.ds(16 * j, 16)]
                for bx in bfly:
                    acc = acc + lax.gather(
                        acc, bx, dnums, (1,),
                        mode=lax.GatherScatterMode.PROMISE_IN_BOUNDS)
                return jnp.where(lanes == (ii - gb), acc, alpha16)
            alpha16 = lax.fori_loop(gb, gb + G, _dot, zero16)
            e16 = jnp.exp(alpha16)
            dst16 = dstv[pl.ds(gb, G)]
            d2v[pl.ds(gb, G)] = N + lax.shift_right_logical(dst16, 7)
            plsc.store_scatter(msg2, [gb + lanes, jnp.bitwise_and(dst16, 127)],
                               e16)

            def _msg(ii, _):
                i = gb + ii
                eb = lax.gather(e16, jnp.full((16, 1), ii, jnp.int32),
                                dnums, (1,),
                                mode=lax.GatherScatterMode.PROMISE_IN_BOUNDS)
                for j in range(D // 16):
                    qr[i, pl.ds(16 * j, 16)] = eb * kvr[i, pl.ds(D + 16 * j, 16)]
                return _
            lax.fori_loop(0, G, _msg, None)
            return _
        lax.fori_loop(0, NGROUP, _group, None)

        pltpu.sync_copy(qr, acc_sh.at[dstv], add=True)
        pltpu.sync_copy(msg2, acc_sh.at[d2v], add=True)

        def _rezero(g, _):
            gb = g * G
            dst16 = dstv[pl.ds(gb, G)]
            plsc.store_scatter(msg2, [gb + lanes, jnp.bitwise_and(dst16, 127)],
                               zero16)
            return _
        lax.fori_loop(0, NGROUP, _rezero, None)

    _prefetch(0, 0)

    def _outer(o, _):
        t = 2 * o
        _prefetch(t + 1, 1)
        _compute_scatter(0)

        @pl.when(t + 2 < NCHUNK)
        def _():
            _prefetch(t + 2, 0)
        _compute_scatter(1)
        return _
    lax.fori_loop(0, NCHUNK // 2, _outer, None)

    plsc.subcore_barrier()
    for t in range(NRC):
        r = row0 + t * RCHUNK

        @pl.when(r < NACC)
        def _():
            pltpu.sync_copy(acc_sh.at[pl.ds(r, RCHUNK)],
                            out_hbm.at[cid, pl.ds(r, RCHUNK)])


@functools.cache
def _edge_kernel():
    mesh = plsc.VectorSubcoreMesh(core_axis_name="c", subcore_axis_name="s")
    return pl.kernel(
        _edge_kernel_body,
        mesh=mesh,
        compiler_params=pltpu.CompilerParams(needs_layout_passes=False),
        out_type=jax.ShapeDtypeStruct((NC, NACC, D), jnp.float32),
        scratch_types=[
            pltpu.VMEM((C,), jnp.int32),
            pltpu.VMEM((C,), jnp.int32),
            pltpu.VMEM((C,), jnp.int32),
            pltpu.VMEM((C,), jnp.int32),
            pltpu.VMEM((C,), jnp.int32),
            pltpu.VMEM((C, D), jnp.float32),
            pltpu.VMEM((C, D), jnp.float32),
            pltpu.VMEM((C, 2 * D), jnp.float32),
            pltpu.VMEM((C, 2 * D), jnp.float32),
            pltpu.VMEM((C, D), jnp.float32),
            pltpu.VMEM_SHARED((NACC, D), jnp.float32),
            pltpu.SemaphoreType.DMA,
            pltpu.SemaphoreType.DMA,
            pltpu.SemaphoreType.DMA,
            pltpu.SemaphoreType.DMA,
        ],
    )


_BN = 1000


def _proj_body(x_ref, w_ref, b_ref, q_ref, kv_ref, s_ref):
    y = jnp.dot(x_ref[...], w_ref[...],
                preferred_element_type=jnp.float32) + b_ref[...]
    q_ref[...] = y[:, :D]
    kv_ref[...] = y[:, D:3 * D]
    s_ref[...] = y[:, 3 * D:]


_PROJ_OUT = [
    jax.ShapeDtypeStruct((N, D), jnp.float32),
    jax.ShapeDtypeStruct((N, 2 * D), jnp.float32),
    jax.ShapeDtypeStruct((N, D), jnp.float32),
]
_PROJ_OUT_SPECS = [
    pl.BlockSpec((_BN, D), lambda i: (i, 0)),
    pl.BlockSpec((_BN, 2 * D), lambda i: (i, 0)),
    pl.BlockSpec((_BN, D), lambda i: (i, 0)),
]


def _proj(x, w, b):
    return pl.pallas_call(
        _proj_body,
        grid=(N // _BN,),
        in_specs=[
            pl.BlockSpec((_BN, D), lambda i: (i, 0)),
            pl.BlockSpec((D, 4 * D), lambda i: (0, 0)),
            pl.BlockSpec((1, 4 * D), lambda i: (0, 0)),
        ],
        out_specs=_PROJ_OUT_SPECS,
        out_shape=_PROJ_OUT,
    )(x, w, b)


def _norm(acc_ref, den_ref):
    num = acc_ref[0] + acc_ref[1]
    den = den_ref[0] + den_ref[1]
    return num / (den + 1e-16)


def _comb_proj_body(acc_ref, den_ref, s_ref, w_ref, b_ref,
                    q_ref, kv_ref, sk_ref):
    h = jnp.maximum(_norm(acc_ref, den_ref) + s_ref[...], 0.0)
    y = jnp.dot(h, w_ref[...], preferred_element_type=jnp.float32) + b_ref[...]
    q_ref[...] = y[:, :D]
    kv_ref[...] = y[:, D:3 * D]
    sk_ref[...] = y[:, 3 * D:]


def _comb_proj(acc, den, s, w, b):
    return pl.pallas_call(
        _comb_proj_body,
        grid=(N // _BN,),
        in_specs=[
            pl.BlockSpec((NC, _BN, D), lambda i: (0, i, 0)),
            pl.BlockSpec((NC, _BN, 1), lambda i: (0, i, 0)),
            pl.BlockSpec((_BN, D), lambda i: (i, 0)),
            pl.BlockSpec((D, 4 * D), lambda i: (0, 0)),
            pl.BlockSpec((1, 4 * D), lambda i: (0, 0)),
        ],
        out_specs=_PROJ_OUT_SPECS,
        out_shape=_PROJ_OUT,
    )(acc, den, s, w, b)


def _final_body(acc_ref, den_ref, s_ref, o_ref):
    o_ref[...] = _norm(acc_ref, den_ref) + s_ref[...]


def _final(acc, den, s):
    return pl.pallas_call(
        _final_body,
        grid=(N // _BN,),
        in_specs=[
            pl.BlockSpec((NC, _BN, D), lambda i: (0, i, 0)),
            pl.BlockSpec((NC, _BN, 1), lambda i: (0, i, 0)),
            pl.BlockSpec((_BN, D), lambda i: (i, 0)),
        ],
        out_specs=pl.BlockSpec((_BN, D), lambda i: (i, 0)),
        out_shape=jax.ShapeDtypeStruct((N, D), jnp.float32),
    )(acc, den, s)


def _split_den(acc):
    den = acc[:, N:N + DROWS, :].reshape(NC, DROWS * D)[:, :N, None]
    return den


def kernel(x, edge_index, Wq1, Wk1, Wv1, Ws1, Wq2, Wk2, Wv2, Ws2,
           bq1, bk1, bv1, bs1, bq2, bk2, bv2, bs2):
    npad = EPAD - E
    src = jnp.concatenate([edge_index[0], jnp.zeros((npad,), jnp.int32)])
    dst = jnp.concatenate(
        [edge_index[1], jnp.full((npad,), PAD_DST, jnp.int32)])
    scale = 1.0 / math.sqrt(D)
    w1 = jnp.concatenate([Wq1.T * scale, Wk1.T, Wv1.T, Ws1.T], axis=1)
    b1 = jnp.concatenate([bq1 * scale, bk1, bv1, bs1])[None, :]
    w2 = jnp.concatenate([Wq2.T * scale, Wk2.T, Wv2.T, Ws2.T], axis=1)
    b2 = jnp.concatenate([bq2 * scale, bk2, bv2, bs2])[None, :]

    qpad = jnp.zeros((NACC - N, D), jnp.float32)
    q1, kv1, s1 = _proj(x, w1, b1)
    acc1 = _edge_kernel()(jnp.concatenate([q1, qpad]), kv1, src, dst)
    q2, kv2, s2 = _comb_proj(acc1, _split_den(acc1), s1, w2, b2)
    acc2 = _edge_kernel()(jnp.concatenate([q2, qpad]), kv2, src, dst)
    return _final(acc2, _split_den(acc2), s2)


# speedup vs baseline: 11.1947x; 1.2831x over previous
import functools
import math

import jax
import jax.numpy as jnp
from jax import lax
from jax.experimental import pallas as pl
from jax.experimental.pallas import tpu as pltpu
from jax.experimental.pallas import tpu_sc as plsc

N = 10000
D = 128
E = 320000

NC = 2
NS = 16
NW = NC * NS
C = 48
NCHUNK = 210
EPW = C * NCHUNK
EPAD = NW * EPW
G = 16
NGROUP = C // G
DROWS = (N + D - 1) // D
NACC = N + DROWS + 1
PAD_DST = NACC - 1
RPT = 640
RCHUNK = 40
NRC = RPT // RCHUNK


def _edge_kernel_body(qk_hbm, v_hbm, src_hbm, dst_hbm, out_hbm,
                      srcv0, srcv1, dstv0, dstv1, d2v,
                      qr0, qr1, kr0, kr1, vr0, vr1, msg2, acc_sh,
                      semq0, semq1, semk0, semk1, semv0, semv1):
    cid = lax.axis_index("c")
    sid = lax.axis_index("s")
    wid = sid * NC + cid
    zero16 = jnp.zeros((16,), jnp.float32)
    lanes = lax.iota(jnp.int32, 16)
    dnums = lax.GatherDimensionNumbers(
        offset_dims=(), collapsed_slice_dims=(0,), start_index_map=(0,))
    bfly = [jnp.reshape(jnp.bitwise_xor(lanes, 1 << k), (16, 1))
            for k in range(4)]
    splat = [jnp.full((16, 1), i, jnp.int32) for i in range(G)]
    srcvs, dstvs = (srcv0, srcv1), (dstv0, dstv1)
    qrs, krs, vrs = (qr0, qr1), (kr0, kr1), (vr0, vr1)
    semqs, semks, semvs = (semq0, semq1), (semk0, semk1), (semv0, semv1)

    def _zrow(r, _):
        for j in range(D // 16):
            msg2[r, pl.ds(16 * j, 16)] = zero16
        return _
    lax.fori_loop(0, C, _zrow, None)
    row0 = sid * RPT
    for t in range(NRC):
        r = row0 + t * RCHUNK

        @pl.when(r < NACC)
        def _():
            pltpu.sync_copy(msg2.at[pl.ds(0, RCHUNK)],
                            acc_sh.at[pl.ds(r, RCHUNK)])

    plsc.subcore_barrier()

    def _prefetch(u, b):
        base = wid * EPW + u * C
        pltpu.sync_copy(src_hbm.at[pl.ds(base, C)], srcvs[b])
        pltpu.sync_copy(dst_hbm.at[pl.ds(base, C)], dstvs[b])
        pltpu.async_copy(qk_hbm.at[dstvs[b]], qrs[b], semqs[b])
        pltpu.async_copy(qk_hbm.at[srcvs[b]], krs[b], semks[b])
        pltpu.async_copy(v_hbm.at[srcvs[b]], vrs[b], semvs[b])

    def _compute_scatter(b):
        qr, kr, vr, dstv = qrs[b], krs[b], vrs[b], dstvs[b]
        pltpu.make_async_copy(qk_hbm.at[dstv], qr, semqs[b]).wait()
        pltpu.make_async_copy(qk_hbm.at[srcvs[b]], kr, semks[b]).wait()
        pltpu.make_async_copy(v_hbm.at[srcvs[b]], vr, semvs[b]).wait()

        def _group(g, _):
            gb = g * G

            def _dot(ii, alpha16):
                i = gb + ii
                acc = zero16
                for j in range(D // 32):
                    q32 = plsc.bitcast(qr[i, pl.ds(16 * j, 16)], jnp.bfloat16)
                    k32 = plsc.bitcast(kr[i, pl.ds(64 + 16 * j, 16)],
                                       jnp.bfloat16)
                    qa, qb = plsc.unpack(q32,
                                         format=plsc.PackFormat.INTERLEAVED)
                    ka, kb = plsc.unpack(k32,
                                         format=plsc.PackFormat.INTERLEAVED)
                    acc = acc + qa * ka + qb * kb
                for bx in bfly:
                    acc = acc + lax.gather(
                        acc, bx, dnums, (1,),
                        mode=lax.GatherScatterMode.PROMISE_IN_BOUNDS)
                return jnp.where(lanes == (ii - gb), acc, alpha16)
            alpha16 = lax.fori_loop(gb, gb + G, _dot, zero16)
            e16 = jnp.exp(alpha16)
            dst16 = dstv[pl.ds(gb, G)]
            d2v[pl.ds(gb, G)] = N + lax.shift_right_logical(dst16, 7)
            plsc.store_scatter(msg2, [gb + lanes, jnp.bitwise_and(dst16, 127)],
                               e16)

            def _msg(ii, _):
                i = gb + ii
                eb = lax.gather(e16, jnp.full((16, 1), ii, jnp.int32),
                                dnums, (1,),
                                mode=lax.GatherScatterMode.PROMISE_IN_BOUNDS)
                for j in range(D // 16):
                    vr[i, pl.ds(16 * j, 16)] = eb * vr[i, pl.ds(16 * j, 16)]
                return _
            lax.fori_loop(0, G, _msg, None)
            return _
        lax.fori_loop(0, NGROUP, _group, None)

        pltpu.sync_copy(vr, acc_sh.at[dstv], add=True)
        pltpu.sync_copy(msg2, acc_sh.at[d2v], add=True)

        def _rezero(g, _):
            gb = g * G
            dst16 = dstv[pl.ds(gb, G)]
            plsc.store_scatter(msg2, [gb + lanes, jnp.bitwise_and(dst16, 127)],
                               zero16)
            return _
        lax.fori_loop(0, NGROUP, _rezero, None)

    _prefetch(0, 0)

    def _outer(o, _):
        t = 2 * o
        _prefetch(t + 1, 1)
        _compute_scatter(0)

        @pl.when(t + 2 < NCHUNK)
        def _():
            _prefetch(t + 2, 0)
        _compute_scatter(1)
        return _
    lax.fori_loop(0, NCHUNK // 2, _outer, None)

    plsc.subcore_barrier()
    for t in range(NRC):
        r = row0 + t * RCHUNK

        @pl.when(r < NACC)
        def _():
            pltpu.sync_copy(acc_sh.at[pl.ds(r, RCHUNK)],
                            out_hbm.at[cid, pl.ds(r, RCHUNK)])


@functools.cache
def _edge_kernel():
    mesh = plsc.VectorSubcoreMesh(core_axis_name="c", subcore_axis_name="s")
    return pl.kernel(
        _edge_kernel_body,
        mesh=mesh,
        compiler_params=pltpu.CompilerParams(needs_layout_passes=False),
        out_type=jax.ShapeDtypeStruct((NC, NACC, D), jnp.float32),
        scratch_types=[
            pltpu.VMEM((C,), jnp.int32),
            pltpu.VMEM((C,), jnp.int32),
            pltpu.VMEM((C,), jnp.int32),
            pltpu.VMEM((C,), jnp.int32),
            pltpu.VMEM((C,), jnp.int32),
            pltpu.VMEM((C, D), jnp.int32),
            pltpu.VMEM((C, D), jnp.int32),
            pltpu.VMEM((C, D), jnp.int32),
            pltpu.VMEM((C, D), jnp.int32),
            pltpu.VMEM((C, D), jnp.float32),
            pltpu.VMEM((C, D), jnp.float32),
            pltpu.VMEM((C, D), jnp.float32),
            pltpu.VMEM_SHARED((NACC, D), jnp.float32),
            pltpu.SemaphoreType.DMA,
            pltpu.SemaphoreType.DMA,
            pltpu.SemaphoreType.DMA,
            pltpu.SemaphoreType.DMA,
            pltpu.SemaphoreType.DMA,
            pltpu.SemaphoreType.DMA,
        ],
    )


_BN = 1000


def _proj_body(x_ref, w_ref, b_ref, q_ref, k_ref, v_ref, s_ref):
    y = jnp.dot(x_ref[...], w_ref[...],
                preferred_element_type=jnp.float32) + b_ref[...]
    q_ref[...] = y[:, :D].astype(jnp.bfloat16)
    k_ref[...] = y[:, D:2 * D].astype(jnp.bfloat16)
    v_ref[...] = y[:, 2 * D:3 * D]
    s_ref[...] = y[:, 3 * D:]


_PROJ_OUT = [
    jax.ShapeDtypeStruct((N, D), jnp.bfloat16),
    jax.ShapeDtypeStruct((N, D), jnp.bfloat16),
    jax.ShapeDtypeStruct((N, D), jnp.float32),
    jax.ShapeDtypeStruct((N, D), jnp.float32),
]
_PROJ_OUT_SPECS = [
    pl.BlockSpec((_BN, D), lambda i: (i, 0)),
    pl.BlockSpec((_BN, D), lambda i: (i, 0)),
    pl.BlockSpec((_BN, D), lambda i: (i, 0)),
    pl.BlockSpec((_BN, D), lambda i: (i, 0)),
]


def _proj(x, w, b):
    return pl.pallas_call(
        _proj_body,
        grid=(N // _BN,),
        in_specs=[
            pl.BlockSpec((_BN, D), lambda i: (i, 0)),
            pl.BlockSpec((D, 4 * D), lambda i: (0, 0)),
            pl.BlockSpec((1, 4 * D), lambda i: (0, 0)),
        ],
        out_specs=_PROJ_OUT_SPECS,
        out_shape=_PROJ_OUT,
    )(x, w, b)


def _norm(acc_ref, den_ref):
    num = acc_ref[0] + acc_ref[1]
    den = den_ref[0] + den_ref[1]
    return num / (den + 1e-16)


def _comb_proj_body(acc_ref, den_ref, s_ref, w_ref, b_ref,
                    q_ref, k_ref, v_ref, sk_ref):
    h = jnp.maximum(_norm(acc_ref, den_ref) + s_ref[...], 0.0)
    y = jnp.dot(h, w_ref[...], preferred_element_type=jnp.float32) + b_ref[...]
    q_ref[...] = y[:, :D].astype(jnp.bfloat16)
    k_ref[...] = y[:, D:2 * D].astype(jnp.bfloat16)
    v_ref[...] = y[:, 2 * D:3 * D]
    sk_ref[...] = y[:, 3 * D:]


def _comb_proj(acc, den, s, w, b):
    return pl.pallas_call(
        _comb_proj_body,
        grid=(N // _BN,),
        in_specs=[
            pl.BlockSpec((NC, _BN, D), lambda i: (0, i, 0)),
            pl.BlockSpec((NC, _BN, 1), lambda i: (0, i, 0)),
            pl.BlockSpec((_BN, D), lambda i: (i, 0)),
            pl.BlockSpec((D, 4 * D), lambda i: (0, 0)),
            pl.BlockSpec((1, 4 * D), lambda i: (0, 0)),
        ],
        out_specs=_PROJ_OUT_SPECS,
        out_shape=_PROJ_OUT,
    )(acc, den, s, w, b)


def _final_body(acc_ref, den_ref, s_ref, o_ref):
    o_ref[...] = _norm(acc_ref, den_ref) + s_ref[...]


def _final(acc, den, s):
    return pl.pallas_call(
        _final_body,
        grid=(N // _BN,),
        in_specs=[
            pl.BlockSpec((NC, _BN, D), lambda i: (0, i, 0)),
            pl.BlockSpec((NC, _BN, 1), lambda i: (0, i, 0)),
            pl.BlockSpec((_BN, D), lambda i: (i, 0)),
        ],
        out_specs=pl.BlockSpec((_BN, D), lambda i: (i, 0)),
        out_shape=jax.ShapeDtypeStruct((N, D), jnp.float32),
    )(acc, den, s)


def _split_den(acc):
    den = acc[:, N:N + DROWS, :].reshape(NC, DROWS * D)[:, :N, None]
    return den


def kernel(x, edge_index, Wq1, Wk1, Wv1, Ws1, Wq2, Wk2, Wv2, Ws2,
           bq1, bk1, bv1, bs1, bq2, bk2, bv2, bs2):
    npad = EPAD - E
    src = jnp.concatenate([edge_index[0], jnp.zeros((npad,), jnp.int32)])
    dst = jnp.concatenate(
        [edge_index[1], jnp.full((npad,), PAD_DST, jnp.int32)])
    scale = 1.0 / math.sqrt(D)
    w1 = jnp.concatenate([Wq1.T * scale, Wk1.T, Wv1.T, Ws1.T], axis=1)
    b1 = jnp.concatenate([bq1 * scale, bk1, bv1, bs1])[None, :]
    w2 = jnp.concatenate([Wq2.T * scale, Wk2.T, Wv2.T, Ws2.T], axis=1)
    b2 = jnp.concatenate([bq2 * scale, bk2, bv2, bs2])[None, :]

    qkpad = jnp.zeros((NACC - N, 2 * D), jnp.bfloat16)

    def _packqk(q, k):
        qk = jnp.concatenate([jnp.concatenate([q, k], axis=1), qkpad])
        return lax.bitcast_convert_type(
            qk.reshape(NACC, D, 2), jnp.int32)

    q1, k1, v1, s1 = _proj(x, w1, b1)
    acc1 = _edge_kernel()(_packqk(q1, k1), v1, src, dst)
    q2, k2, v2, s2 = _comb_proj(acc1, _split_den(acc1), s1, w2, b2)
    acc2 = _edge_kernel()(_packqk(q2, k2), v2, src, dst)
    return _final(acc2, _split_den(acc2), s2)
